# baseline (speedup 1.0000x reference)
import functools

import jax
import jax.numpy as jnp
from jax import lax
from jax.experimental import pallas as pl
from jax.experimental.pallas import tpu as pltpu
from jax.experimental.pallas import tpu_sc as plsc

N_NODES = 10000
N_EDGES = 160000
D = 256

NC = 2
NS = 16
L = 16
NW = NC * NS

BE = 80
BBLK = N_EDGES // NS // BE
DR = 64
DC = 256
DRT = DR // NS

ROWS_PER_W = 320
CH = 4000
NCH = N_EDGES // CH
CE = 64

MM_BLK = 1000


def _mm_body(x_ref, w_ref, o_ref):
    o_ref[...] = jnp.dot(x_ref[...], w_ref[...],
                         preferred_element_type=jnp.float32)


def _matmul(x, W):
    return pl.pallas_call(
        _mm_body,
        grid=(N_NODES // MM_BLK,),
        in_specs=[
            pl.BlockSpec((MM_BLK, D), lambda i: (i, 0)),
            pl.BlockSpec((D, D), lambda i: (0, 0)),
        ],
        out_specs=pl.BlockSpec((MM_BLK, D), lambda i: (i, 0)),
        out_shape=jax.ShapeDtypeStruct((N_NODES, D), jnp.float32),
    )(x, W)


_mesh_b = plsc.VectorSubcoreMesh(core_axis_name="c", subcore_axis_name="s",
                                 num_cores=1, num_subcores=NS)


@functools.partial(
    pl.kernel,
    out_type=jax.ShapeDtypeStruct((NS, BBLK, BE), jnp.float32),
    mesh=_mesh_b,
    scratch_types=[
        pltpu.VMEM((BBLK, BE), jnp.int32),
        pltpu.VMEM((BBLK, BE), jnp.int32),
        pltpu.VMEM((BBLK, BE), jnp.float32),
        pltpu.VMEM((DR, DC), jnp.float32),
        pltpu.VMEM((DRT, DC), jnp.float32),
        pltpu.VMEM((DRT, DC), jnp.float32),
        pltpu.VMEM_SHARED((NS, DR, DC), jnp.float32),
        pltpu.VMEM_SHARED((DR, DC), jnp.float32),
    ],
    compiler_params=pltpu.CompilerParams(needs_layout_passes=False),
)
def _sc_norm(src_hbm, dst_hbm, ew_hbm, nrm_hbm,
             src2, dst2, ew2, dis_v, dtmp, htmp, sh_hists, sh_dis):
    s = lax.axis_index("s")
    zeros = jnp.zeros((L,), jnp.float32)

    pltpu.sync_copy(src_hbm.at[s], src2)
    pltpu.sync_copy(dst_hbm.at[s], dst2)
    pltpu.sync_copy(ew_hbm.at[s], ew2)

    def _zhist(r, _):
        for j in range(DC // L):
            dis_v[r, pl.ds(j * L, L)] = zeros
        return 0
    lax.fori_loop(0, DR, _zhist, 0)

    def _deg(g, _):
        for j in range(BE // L):
            dv = dst2[g, pl.ds(j * L, L)]
            ev = ew2[g, pl.ds(j * L, L)]
            plsc.addupdate_scatter(dis_v, [dv >> 8, dv & 255], ev)
        return 0
    lax.fori_loop(0, BBLK, _deg, 0)
    pltpu.sync_copy(dis_v, sh_hists.at[s])
    plsc.subcore_barrier()

    pltpu.sync_copy(sh_hists.at[0, pl.ds(s * DRT, DRT)], dtmp)
    for p in range(1, NS):
        pltpu.sync_copy(sh_hists.at[p, pl.ds(s * DRT, DRT)], htmp)
        def _accum(r, _):
            for j in range(DC // L):
                dtmp[r, pl.ds(j * L, L)] = (dtmp[r, pl.ds(j * L, L)]
                                            + htmp[r, pl.ds(j * L, L)])
            return 0
        lax.fori_loop(0, DRT, _accum, 0)

    def _rsqrt(k, _):
        r = k // (DC // L)
        j16 = (k % (DC // L)) * L
        d = dtmp[r, pl.ds(j16, L)]
        dp = jnp.where(d > 0.0, d, 1.0)
        s0 = 0.5 * (1.0 + dp)
        def _nw(_i, s_c):
            return 0.5 * (s_c + dp / s_c)
        s0 = lax.fori_loop(0, 30, _nw, s0)
        dtmp[r, pl.ds(j16, L)] = jnp.where(d > 0.0, 1.0 / s0, 0.0)
        return 0
    lax.fori_loop(0, DRT * DC // L, _rsqrt, 0)
    pltpu.sync_copy(dtmp, sh_dis.at[pl.ds(s * DRT, DRT)])
    plsc.subcore_barrier()

    pltpu.sync_copy(sh_dis, dis_v)

    def _norm(g, _):
        for j in range(BE // L):
            sv = src2[g, pl.ds(j * L, L)]
            dv = dst2[g, pl.ds(j * L, L)]
            ev = ew2[g, pl.ds(j * L, L)]
            nm = plsc.load_gather(dis_v, [sv >> 8, sv & 255]) * ev \
                * plsc.load_gather(dis_v, [dv >> 8, dv & 255])
            ew2[g, pl.ds(j * L, L)] = nm
        return 0
    lax.fori_loop(0, BBLK, _norm, 0)
    pltpu.sync_copy(ew2, nrm_hbm.at[s])


_mesh_c = plsc.VectorSubcoreMesh(core_axis_name="c", subcore_axis_name="s",
                                 num_cores=NC, num_subcores=NS)


@functools.partial(
    pl.kernel,
    out_type=jax.ShapeDtypeStruct((N_NODES, D), jnp.float32),
    mesh=_mesh_c,
    scratch_types=[
        pltpu.VMEM((CH + CE,), jnp.int32),
        pltpu.VMEM((CH + CE,), jnp.int32),
        pltpu.VMEM((CH + CE,), jnp.float32),
        pltpu.VMEM((2, CE), jnp.int32),
        pltpu.VMEM((CE, D), jnp.float32),
        pltpu.VMEM((CE, D), jnp.float32),
        pltpu.VMEM((ROWS_PER_W, D), jnp.float32),
        pltpu.SemaphoreType.DMA,
        pltpu.SemaphoreType.DMA,
    ],
    compiler_params=pltpu.CompilerParams(needs_layout_passes=False),
)
def _sc_scatter(src_hbm, dst_hbm, nrm_hbm, h_hbm, out_hbm,
                dstc, srcc, nmc, bidx2, rows0, rows1, acc,
                sem0, sem1):
    c = lax.axis_index("c")
    s = lax.axis_index("s")
    w = c * NS + s
    wlo = w * ROWS_PER_W
    zeros = jnp.zeros((L,), jnp.float32)
    zeros_i = jnp.zeros((L,), jnp.int32)
    iota = lax.iota(jnp.int32, L)

    def _zacc(r, _):
        for j in range(D // L):
            acc[r, pl.ds(j * L, L)] = zeros
        return 0
    lax.fori_loop(0, ROWS_PER_W, _zacc, 0)

    def _chunk(k, _):
        base = k * CH
        pltpu.sync_copy(dst_hbm.at[pl.ds(base, CH)], dstc.at[pl.ds(0, CH)])
        pltpu.sync_copy(src_hbm.at[pl.ds(base, CH)], srcc.at[pl.ds(0, CH)])
        pltpu.sync_copy(nrm_hbm.at[pl.ds(base, CH)], nmc.at[pl.ds(0, CH)])

        def _scan(g, cnt):
            dv = dstc[pl.ds(g * L, L)]
            own = ((dv * 6554) >> 21) == w
            plsc.store_compressed(srcc.at[pl.ds(cnt, L)],
                                  srcc[pl.ds(g * L, L)], mask=own)
            plsc.store_compressed(dstc.at[pl.ds(cnt, L)], dv - wlo, mask=own)
            plsc.store_compressed(nmc.at[pl.ds(cnt, L)],
                                  nmc[pl.ds(g * L, L)], mask=own)
            return cnt + jnp.sum(own.astype(jnp.int32))
        cnt = lax.fori_loop(0, CH // L, _scan, jnp.int32(0))

        for t in range(CE // L):
            srcc[pl.ds(cnt + t * L, L)] = zeros_i
            dstc[pl.ds(cnt + t * L, L)] = zeros_i
            nmc[pl.ds(cnt + t * L, L)] = zeros
        nb = (cnt + CE - 1) // CE

        def _stage_issue(b, par, rows, sem):
            for t in range(CE // L):
                bidx2[par, pl.ds(t * L, L)] = srcc[pl.ds(b * CE + t * L, L)]
            pltpu.async_copy(h_hbm.at[bidx2.at[par]], rows, sem)

        def _process(b, rows):
            for q in range(CE // L):
                dlv = dstc[pl.ds(b * CE + q * L, L)]
                nmv = nmc[pl.ds(b * CE + q * L, L)]
                rq = iota + (q * L)
                def _cols(cc, _3):
                    bc0 = lax.broadcast(cc * L, (L,))
                    for r in range(L):
                        colv = bc0 + ((iota + r) & (L - 1))
                        vals = plsc.load_gather(rows, [rq, colv])
                        plsc.addupdate_scatter(acc, [dlv, colv], vals * nmv)
                    return 0
                lax.fori_loop(0, D // L, _cols, 0)

        @pl.when(nb > 0)
        def _():
            _stage_issue(0, 0, rows0, sem0)

        def _pair(pp, _2):
            for par in range(2):
                b = pp * 2 + par
                rows, sem = (rows0, sem0) if par == 0 else (rows1, sem1)
                npar = 1 - par
                nrows, nsem = (rows1, sem1) if par == 0 else (rows0, sem0)
                @pl.when(b < nb)
                def _():
                    pltpu.make_async_copy(
                        h_hbm.at[bidx2.at[par]], rows, sem).wait()
                    @pl.when(b + 1 < nb)
                    def _():
                        _stage_issue(b + 1, npar, nrows, nsem)
                    _process(b, rows)
            return 0
        lax.fori_loop(0, (nb + 1) // 2, _pair, 0)
        return 0
    lax.fori_loop(0, NCH, _chunk, 0)

    def _relu(r, _):
        for j in range(D // L):
            v = acc[r, pl.ds(j * L, L)]
            acc[r, pl.ds(j * L, L)] = jnp.maximum(v, 0.0) + v
        return 0
    lax.fori_loop(0, ROWS_PER_W, _relu, 0)

    @pl.when(w < NW - 1)
    def _():
        pltpu.sync_copy(acc, out_hbm.at[pl.ds(wlo, ROWS_PER_W)])

    @pl.when(w == NW - 1)
    def _():
        last = N_NODES - (NW - 1) * ROWS_PER_W
        pltpu.sync_copy(acc.at[pl.ds(0, last)],
                        out_hbm.at[pl.ds(wlo, last)])


def kernel(x, edge_index, edge_weights, W):
    src = edge_index[0].astype(jnp.int32)
    dst = edge_index[1].astype(jnp.int32)
    ew = edge_weights.astype(jnp.float32)

    h = _matmul(x, W)
    nrm = _sc_norm(src.reshape(NS, BBLK, BE), dst.reshape(NS, BBLK, BE),
                   ew.reshape(NS, BBLK, BE))
    return _sc_scatter(src, dst, nrm.reshape(-1), h)


# speedup vs baseline: 2.1593x; 2.1593x over previous
import functools

import jax
import jax.numpy as jnp
from jax import lax
from jax.experimental import pallas as pl
from jax.experimental.pallas import tpu as pltpu
from jax.experimental.pallas import tpu_sc as plsc

N_NODES = 10000
N_EDGES = 160000
D = 256

NC = 2
NS = 16
L = 16
NW = NC * NS

BE = 80
BBLK = N_EDGES // NS // BE
DR = 64
DC = 256
DRT = DR // NS

ROWS_PER_W = 320
CH = 2000
NCH = N_EDGES // CH
CE = 128
DP = D // 2

MM_BLK = 1000


def _mm_body(x_ref, w_ref, o_ref):
    o = jnp.dot(x_ref[...], w_ref[...], preferred_element_type=jnp.float32)
    o_ref[...] = o.astype(jnp.bfloat16)


def _matmul(x, W):
    return pl.pallas_call(
        _mm_body,
        grid=(N_NODES // MM_BLK,),
        in_specs=[
            pl.BlockSpec((MM_BLK, D), lambda i: (i, 0)),
            pl.BlockSpec((D, D), lambda i: (0, 0)),
        ],
        out_specs=pl.BlockSpec((MM_BLK, D), lambda i: (i, 0)),
        out_shape=jax.ShapeDtypeStruct((N_NODES, D), jnp.bfloat16),
    )(x, W)


_mesh_b = plsc.VectorSubcoreMesh(core_axis_name="c", subcore_axis_name="s",
                                 num_cores=1, num_subcores=NS)


@functools.partial(
    pl.kernel,
    out_type=jax.ShapeDtypeStruct((NS, BBLK, BE), jnp.float32),
    mesh=_mesh_b,
    scratch_types=[
        pltpu.VMEM((BBLK, BE), jnp.int32),
        pltpu.VMEM((BBLK, BE), jnp.int32),
        pltpu.VMEM((BBLK, BE), jnp.float32),
        pltpu.VMEM((DR, DC), jnp.float32),
        pltpu.VMEM((DRT, DC), jnp.float32),
        pltpu.VMEM((DRT, DC), jnp.float32),
        pltpu.VMEM_SHARED((NS, DR, DC), jnp.float32),
        pltpu.VMEM_SHARED((DR, DC), jnp.float32),
    ],
    compiler_params=pltpu.CompilerParams(needs_layout_passes=False),
)
def _sc_norm(src_hbm, dst_hbm, ew_hbm, nrm_hbm,
             src2, dst2, ew2, dis_v, dtmp, htmp, sh_hists, sh_dis):
    s = lax.axis_index("s")
    zeros = jnp.zeros((L,), jnp.float32)

    pltpu.sync_copy(src_hbm.at[s], src2)
    pltpu.sync_copy(dst_hbm.at[s], dst2)
    pltpu.sync_copy(ew_hbm.at[s], ew2)

    def _zhist(r, _):
        for j in range(DC // L):
            dis_v[r, pl.ds(j * L, L)] = zeros
        return 0
    lax.fori_loop(0, DR, _zhist, 0)

    def _deg(g, _):
        for j in range(BE // L):
            dv = dst2[g, pl.ds(j * L, L)]
            ev = ew2[g, pl.ds(j * L, L)]
            plsc.addupdate_scatter(dis_v, [dv >> 8, dv & 255], ev)
        return 0
    lax.fori_loop(0, BBLK, _deg, 0)
    pltpu.sync_copy(dis_v, sh_hists.at[s])
    plsc.subcore_barrier()

    pltpu.sync_copy(sh_hists.at[0, pl.ds(s * DRT, DRT)], dtmp)
    for p in range(1, NS):
        pltpu.sync_copy(sh_hists.at[p, pl.ds(s * DRT, DRT)], htmp)
        def _accum(r, _):
            for j in range(DC // L):
                dtmp[r, pl.ds(j * L, L)] = (dtmp[r, pl.ds(j * L, L)]
                                            + htmp[r, pl.ds(j * L, L)])
            return 0
        lax.fori_loop(0, DRT, _accum, 0)

    def _rsqrt(k, _):
        r = k // (DC // L)
        j16 = (k % (DC // L)) * L
        d = dtmp[r, pl.ds(j16, L)]
        dp = jnp.where(d > 0.0, d, 1.0)
        s0 = 0.5 * (1.0 + dp)
        def _nw(_i, s_c):
            return 0.5 * (s_c + dp / s_c)
        s0 = lax.fori_loop(0, 30, _nw, s0)
        dtmp[r, pl.ds(j16, L)] = jnp.where(d > 0.0, 1.0 / s0, 0.0)
        return 0
    lax.fori_loop(0, DRT * DC // L, _rsqrt, 0)
    pltpu.sync_copy(dtmp, sh_dis.at[pl.ds(s * DRT, DRT)])
    plsc.subcore_barrier()

    pltpu.sync_copy(sh_dis, dis_v)

    def _norm(g, _):
        for j in range(BE // L):
            sv = src2[g, pl.ds(j * L, L)]
            dv = dst2[g, pl.ds(j * L, L)]
            ev = ew2[g, pl.ds(j * L, L)]
            nm = plsc.load_gather(dis_v, [sv >> 8, sv & 255]) * ev \
                * plsc.load_gather(dis_v, [dv >> 8, dv & 255])
            ew2[g, pl.ds(j * L, L)] = nm
        return 0
    lax.fori_loop(0, BBLK, _norm, 0)
    pltpu.sync_copy(ew2, nrm_hbm.at[s])


_mesh_c = plsc.VectorSubcoreMesh(core_axis_name="c", subcore_axis_name="s",
                                 num_cores=NC, num_subcores=NS)


@functools.partial(
    pl.kernel,
    out_type=jax.ShapeDtypeStruct((N_NODES, D), jnp.float32),
    mesh=_mesh_c,
    scratch_types=[
        pltpu.VMEM((CH,), jnp.int32),
        pltpu.VMEM((CH,), jnp.int32),
        pltpu.VMEM((CH,), jnp.float32),
        pltpu.VMEM((CH + CE,), jnp.int32),
        pltpu.VMEM((CH + CE,), jnp.int32),
        pltpu.VMEM((CH + CE,), jnp.float32),
        pltpu.VMEM((1, CE), jnp.int32),
        pltpu.VMEM((CE, DP), jnp.int32),
        pltpu.VMEM((ROWS_PER_W, D), jnp.float32),
        pltpu.SemaphoreType.DMA,
    ],
    compiler_params=pltpu.CompilerParams(needs_layout_passes=False),
)
def _sc_scatter(src_hbm, dst_hbm, nrm_hbm, h_hbm, out_hbm,
                dstc, srcc, nmc, csrc, cdl, cnm, bidx2, rows0, acc, sem0):
    c = lax.axis_index("c")
    s = lax.axis_index("s")
    w = c * NS + s
    wlo = w * ROWS_PER_W
    zeros = jnp.zeros((L,), jnp.float32)
    zeros_i = jnp.zeros((L,), jnp.int32)
    iota = lax.iota(jnp.int32, L)

    def _zacc(r, _):
        for j in range(D // L):
            acc[r, pl.ds(j * L, L)] = zeros
        return 0
    lax.fori_loop(0, ROWS_PER_W, _zacc, 0)

    def _stage_issue(blk, par, rows, sem):
        for t in range(CE // L):
            bidx2[par, pl.ds(t * L, L)] = csrc[pl.ds(blk * CE + t * L, L)]
        pltpu.async_copy(h_hbm.at[bidx2.at[par]], rows, sem)

    def _process(blk, rows):
        def _qgrp(q, _4):
            dlv = cdl[pl.ds(blk * CE + q * L, L)]
            nmv = cnm[pl.ds(blk * CE + q * L, L)]
            rq = iota + (q * L)
            def _cols(cc, _3):
                bc0 = lax.broadcast(cc * L, (L,))
                for r in range(L):
                    colv = bc0 + ((iota + r) & (L - 1))
                    word = plsc.load_gather(rows, [rq, colv])
                    va = plsc.bitcast(word << 16, jnp.float32)
                    vb = plsc.bitcast(word & jnp.int32(-65536), jnp.float32)
                    oc = colv * 2
                    plsc.addupdate_scatter(acc, [dlv, oc], va * nmv)
                    plsc.addupdate_scatter(acc, [dlv, oc + 1], vb * nmv)
                return 0
            lax.fori_loop(0, DP // L, _cols, 0)
            return 0
        lax.fori_loop(0, CE // L, _qgrp, 0)

    def _chunk(k, cnt0):
        def _load_scan(c0):
            base = k * CH
            pltpu.sync_copy(dst_hbm.at[pl.ds(base, CH)], dstc)
            pltpu.sync_copy(src_hbm.at[pl.ds(base, CH)], srcc)
            pltpu.sync_copy(nrm_hbm.at[pl.ds(base, CH)], nmc)

            def _scan(g, cnt):
                dv = dstc[pl.ds(g * L, L)]
                own = ((dv * 6554) >> 21) == w
                plsc.store_compressed(csrc.at[pl.ds(cnt, L)],
                                      srcc[pl.ds(g * L, L)], mask=own)
                plsc.store_compressed(cdl.at[pl.ds(cnt, L)], dv - wlo,
                                      mask=own)
                plsc.store_compressed(cnm.at[pl.ds(cnt, L)],
                                      nmc[pl.ds(g * L, L)], mask=own)
                return cnt + jnp.sum(own.astype(jnp.int32))
            return lax.fori_loop(0, CH // L, _scan, c0)

        def _tail_pad(c0):
            for t in range(CE // L):
                csrc[pl.ds(c0 + t * L, L)] = zeros_i
                cdl[pl.ds(c0 + t * L, L)] = zeros_i
                cnm[pl.ds(c0 + t * L, L)] = zeros
            return jnp.where(c0 > 0, jnp.int32(CE), jnp.int32(0))

        cnt = lax.cond(k < NCH, _load_scan, _tail_pad, cnt0)
        nbf = cnt // CE

        def _blk(b, _2):
            _stage_issue(b, 0, rows0, sem0)
            pltpu.make_async_copy(h_hbm.at[bidx2.at[0]], rows0, sem0).wait()
            _process(b, rows0)
            return 0
        lax.fori_loop(0, nbf, _blk, 0)

        lo = cnt - nbf * CE
        @pl.when(nbf > 0)
        def _():
            for t in range(CE // L):
                @pl.when(t * L < lo)
                def _():
                    off = nbf * CE + t * L
                    csrc[pl.ds(t * L, L)] = csrc[pl.ds(off, L)]
                    cdl[pl.ds(t * L, L)] = cdl[pl.ds(off, L)]
                    cnm[pl.ds(t * L, L)] = cnm[pl.ds(off, L)]
        return lo
    lax.fori_loop(0, NCH + 1, _chunk, jnp.int32(0))

    def _relu(r, _):
        for j in range(D // L):
            v = acc[r, pl.ds(j * L, L)]
            acc[r, pl.ds(j * L, L)] = jnp.maximum(v, 0.0) + v
        return 0
    lax.fori_loop(0, ROWS_PER_W, _relu, 0)

    @pl.when(w < NW - 1)
    def _():
        pltpu.sync_copy(acc, out_hbm.at[pl.ds(wlo, ROWS_PER_W)])

    @pl.when(w == NW - 1)
    def _():
        last = N_NODES - (NW - 1) * ROWS_PER_W
        pltpu.sync_copy(acc.at[pl.ds(0, last)],
                        out_hbm.at[pl.ds(wlo, last)])


def kernel(x, edge_index, edge_weights, W):
    src = edge_index[0].astype(jnp.int32)
    dst = edge_index[1].astype(jnp.int32)
    ew = edge_weights.astype(jnp.float32)

    h = _matmul(x, W)
    hp = jax.lax.bitcast_convert_type(h.reshape(N_NODES, D // 2, 2),
                                      jnp.int32)
    nrm = _sc_norm(src.reshape(NS, BBLK, BE), dst.reshape(NS, BBLK, BE),
                   ew.reshape(NS, BBLK, BE))
    return _sc_scatter(src, dst, nrm.reshape(-1), hp)


# speedup vs baseline: 2.3347x; 1.0812x over previous
import functools

import jax
import jax.numpy as jnp
from jax import lax
from jax.experimental import pallas as pl
from jax.experimental.pallas import tpu as pltpu
from jax.experimental.pallas import tpu_sc as plsc

N_NODES = 10000
N_EDGES = 160000
D = 256

NC = 2
NS = 16
L = 16
NW = NC * NS

BE = 80
BBLK = N_EDGES // NS // BE
DR = 64
DC = 256
DRT = DR // NS

ROWS_PER_W = 320
CH = 4000
NCH = N_EDGES // CH
CE = 128
DP = D // 2

MM_BLK = 1000


def _mm_body(x_ref, w_ref, o_ref):
    o = jnp.dot(x_ref[...], w_ref[...], preferred_element_type=jnp.float32)
    o_ref[...] = o.astype(jnp.bfloat16)


def _matmul(x, W):
    return pl.pallas_call(
        _mm_body,
        grid=(N_NODES // MM_BLK,),
        in_specs=[
            pl.BlockSpec((MM_BLK, D), lambda i: (i, 0)),
            pl.BlockSpec((D, D), lambda i: (0, 0)),
        ],
        out_specs=pl.BlockSpec((MM_BLK, D), lambda i: (i, 0)),
        out_shape=jax.ShapeDtypeStruct((N_NODES, D), jnp.bfloat16),
    )(x, W)


_mesh_b = plsc.VectorSubcoreMesh(core_axis_name="c", subcore_axis_name="s",
                                 num_cores=1, num_subcores=NS)


@functools.partial(
    pl.kernel,
    out_type=jax.ShapeDtypeStruct((NS, BBLK, BE), jnp.float32),
    mesh=_mesh_b,
    scratch_types=[
        pltpu.VMEM((BBLK, BE), jnp.int32),
        pltpu.VMEM((BBLK, BE), jnp.int32),
        pltpu.VMEM((BBLK, BE), jnp.float32),
        pltpu.VMEM((DR, DC), jnp.float32),
        pltpu.VMEM((DRT, DC), jnp.float32),
        pltpu.VMEM((DRT, DC), jnp.float32),
        pltpu.VMEM_SHARED((NS, DR, DC), jnp.float32),
        pltpu.VMEM_SHARED((DR, DC), jnp.float32),
    ],
    compiler_params=pltpu.CompilerParams(needs_layout_passes=False),
)
def _sc_norm(src_hbm, dst_hbm, ew_hbm, nrm_hbm,
             src2, dst2, ew2, dis_v, dtmp, htmp, sh_hists, sh_dis):
    s = lax.axis_index("s")
    zeros = jnp.zeros((L,), jnp.float32)

    pltpu.sync_copy(src_hbm.at[s], src2)
    pltpu.sync_copy(dst_hbm.at[s], dst2)
    pltpu.sync_copy(ew_hbm.at[s], ew2)

    def _zhist(r, _):
        for j in range(DC // L):
            dis_v[r, pl.ds(j * L, L)] = zeros
        return 0
    lax.fori_loop(0, DR, _zhist, 0)

    def _deg(g, _):
        for j in range(BE // L):
            dv = dst2[g, pl.ds(j * L, L)]
            ev = ew2[g, pl.ds(j * L, L)]
            plsc.addupdate_scatter(dis_v, [dv >> 8, dv & 255], ev)
        return 0
    lax.fori_loop(0, BBLK, _deg, 0)
    pltpu.sync_copy(dis_v, sh_hists.at[s])
    plsc.subcore_barrier()

    pltpu.sync_copy(sh_hists.at[0, pl.ds(s * DRT, DRT)], dtmp)
    for p in range(1, NS):
        pltpu.sync_copy(sh_hists.at[p, pl.ds(s * DRT, DRT)], htmp)
        def _accum(r, _):
            for j in range(DC // L):
                dtmp[r, pl.ds(j * L, L)] = (dtmp[r, pl.ds(j * L, L)]
                                            + htmp[r, pl.ds(j * L, L)])
            return 0
        lax.fori_loop(0, DRT, _accum, 0)

    def _rsqrt(k, _):
        r = k // (DC // L)
        j16 = (k % (DC // L)) * L
        d = dtmp[r, pl.ds(j16, L)]
        dp = jnp.where(d > 0.0, d, 1.0)
        s0 = 0.5 * (1.0 + dp)
        def _nw(_i, s_c):
            return 0.5 * (s_c + dp / s_c)
        s0 = lax.fori_loop(0, 30, _nw, s0)
        dtmp[r, pl.ds(j16, L)] = jnp.where(d > 0.0, 1.0 / s0, 0.0)
        return 0
    lax.fori_loop(0, DRT * DC // L, _rsqrt, 0)
    pltpu.sync_copy(dtmp, sh_dis.at[pl.ds(s * DRT, DRT)])
    plsc.subcore_barrier()

    pltpu.sync_copy(sh_dis, dis_v)

    def _norm(g, _):
        for j in range(BE // L):
            sv = src2[g, pl.ds(j * L, L)]
            dv = dst2[g, pl.ds(j * L, L)]
            ev = ew2[g, pl.ds(j * L, L)]
            nm = plsc.load_gather(dis_v, [sv >> 8, sv & 255]) * ev \
                * plsc.load_gather(dis_v, [dv >> 8, dv & 255])
            ew2[g, pl.ds(j * L, L)] = nm
        return 0
    lax.fori_loop(0, BBLK, _norm, 0)
    pltpu.sync_copy(ew2, nrm_hbm.at[s])


_mesh_c = plsc.VectorSubcoreMesh(core_axis_name="c", subcore_axis_name="s",
                                 num_cores=NC, num_subcores=NS)


@functools.partial(
    pl.kernel,
    out_type=jax.ShapeDtypeStruct((N_NODES, D), jnp.float32),
    mesh=_mesh_c,
    scratch_types=[
        pltpu.VMEM((CH,), jnp.int32),
        pltpu.VMEM((CH,), jnp.int32),
        pltpu.VMEM((CH,), jnp.float32),
        pltpu.VMEM((CH + CE,), jnp.int32),
        pltpu.VMEM((CH + CE,), jnp.int32),
        pltpu.VMEM((CH + CE,), jnp.float32),
        pltpu.VMEM((1, CE), jnp.int32),
        pltpu.VMEM((CE, DP), jnp.int32),
        pltpu.VMEM((ROWS_PER_W, D), jnp.float32),
        pltpu.SemaphoreType.DMA,
    ],
    compiler_params=pltpu.CompilerParams(needs_layout_passes=False),
)
def _sc_scatter(src_hbm, dst_hbm, nrm_hbm, h_hbm, out_hbm,
                dstc, srcc, nmc, csrc, cdl, cnm, bidx2, rows0, acc, sem0):
    c = lax.axis_index("c")
    s = lax.axis_index("s")
    w = c * NS + s
    wlo = w * ROWS_PER_W
    zeros = jnp.zeros((L,), jnp.float32)
    zeros_i = jnp.zeros((L,), jnp.int32)
    iota = lax.iota(jnp.int32, L)

    def _zacc(r, _):
        for j in range(D // L):
            acc[r, pl.ds(j * L, L)] = zeros
        return 0
    lax.fori_loop(0, ROWS_PER_W, _zacc, 0)

    def _stage_issue(blk, par, rows, sem):
        for t in range(CE // L):
            bidx2[par, pl.ds(t * L, L)] = csrc[pl.ds(blk * CE + t * L, L)]
        pltpu.async_copy(h_hbm.at[bidx2.at[par]], rows, sem)

    def _process(blk, rows):
        def _qgrp(q, _4):
            dlv = cdl[pl.ds(blk * CE + q * L, L)]
            nmv = cnm[pl.ds(blk * CE + q * L, L)]
            rq = iota + (q * L)
            def _cols(cc, _3):
                bc0 = lax.broadcast(cc * L, (L,))
                for r in range(L):
                    colv = bc0 + ((iota + r) & (L - 1))
                    word = plsc.load_gather(rows, [rq, colv])
                    va = plsc.bitcast(word << 16, jnp.float32)
                    vb = plsc.bitcast(word & jnp.int32(-65536), jnp.float32)
                    oc = colv * 2
                    plsc.addupdate_scatter(acc, [dlv, oc], va * nmv)
                    plsc.addupdate_scatter(acc, [dlv, oc + 1], vb * nmv)
                return 0
            lax.fori_loop(0, DP // L, _cols, 0)
            return 0
        lax.fori_loop(0, CE // L, _qgrp, 0)

    def _chunk(k, cnt0):
        def _load_scan(c0):
            base = k * CH
            pltpu.sync_copy(dst_hbm.at[pl.ds(base, CH)], dstc)
            pltpu.sync_copy(src_hbm.at[pl.ds(base, CH)], srcc)
            pltpu.sync_copy(nrm_hbm.at[pl.ds(base, CH)], nmc)

            def _scan(g, cnt):
                dv = dstc[pl.ds(g * L, L)]
                own = ((dv * 6554) >> 21) == w
                plsc.store_compressed(csrc.at[pl.ds(cnt, L)],
                                      srcc[pl.ds(g * L, L)], mask=own)
                plsc.store_compressed(cdl.at[pl.ds(cnt, L)], dv - wlo,
                                      mask=own)
                plsc.store_compressed(cnm.at[pl.ds(cnt, L)],
                                      nmc[pl.ds(g * L, L)], mask=own)
                pc = plsc.all_reduce_population_count(own)
                return cnt + pc[0]
            return lax.fori_loop(0, CH // L, _scan, c0)

        def _tail_pad(c0):
            for t in range(CE // L):
                csrc[pl.ds(c0 + t * L, L)] = zeros_i
                cdl[pl.ds(c0 + t * L, L)] = zeros_i
                cnm[pl.ds(c0 + t * L, L)] = zeros
            return jnp.where(c0 > 0, jnp.int32(CE), jnp.int32(0))

        cnt = lax.cond(k < NCH, _load_scan, _tail_pad, cnt0)
        nbf = cnt // CE

        def _blk(b, _2):
            _stage_issue(b, 0, rows0, sem0)
            pltpu.make_async_copy(h_hbm.at[bidx2.at[0]], rows0, sem0).wait()
            _process(b, rows0)
            return 0
        lax.fori_loop(0, nbf, _blk, 0)

        lo = cnt - nbf * CE
        @pl.when(nbf > 0)
        def _():
            for t in range(CE // L):
                @pl.when(t * L < lo)
                def _():
                    off = nbf * CE + t * L
                    csrc[pl.ds(t * L, L)] = csrc[pl.ds(off, L)]
                    cdl[pl.ds(t * L, L)] = cdl[pl.ds(off, L)]
                    cnm[pl.ds(t * L, L)] = cnm[pl.ds(off, L)]
        return lo
    lax.fori_loop(0, NCH + 1, _chunk, jnp.int32(0))

    def _relu(r, _):
        for j in range(D // L):
            v = acc[r, pl.ds(j * L, L)]
            acc[r, pl.ds(j * L, L)] = jnp.maximum(v, 0.0) + v
        return 0
    lax.fori_loop(0, ROWS_PER_W, _relu, 0)

    @pl.when(w < NW - 1)
    def _():
        pltpu.sync_copy(acc, out_hbm.at[pl.ds(wlo, ROWS_PER_W)])

    @pl.when(w == NW - 1)
    def _():
        last = N_NODES - (NW - 1) * ROWS_PER_W
        pltpu.sync_copy(acc.at[pl.ds(0, last)],
                        out_hbm.at[pl.ds(wlo, last)])


def kernel(x, edge_index, edge_weights, W):
    src = edge_index[0].astype(jnp.int32)
    dst = edge_index[1].astype(jnp.int32)
    ew = edge_weights.astype(jnp.float32)

    h = _matmul(x, W)
    hp = jax.lax.bitcast_convert_type(h.reshape(N_NODES, D // 2, 2),
                                      jnp.int32)
    nrm = _sc_norm(src.reshape(NS, BBLK, BE), dst.reshape(NS, BBLK, BE),
                   ew.reshape(NS, BBLK, BE))
    return _sc_scatter(src, dst, nrm.reshape(-1), hp)


# speedup vs baseline: 2.4772x; 1.0610x over previous
import functools

import jax
import jax.numpy as jnp
from jax import lax
from jax.experimental import pallas as pl
from jax.experimental.pallas import tpu as pltpu
from jax.experimental.pallas import tpu_sc as plsc

N_NODES = 10000
N_EDGES = 160000
D = 256

NC = 2
NS = 16
L = 16
NW = NC * NS

BE = 80
BBLK = N_EDGES // NS // BE
DR = 64
DC = 256
DRT = DR // NS

ROWS_PER_W = 320
CH = 4000
NCH = N_EDGES // CH
CE = 128
DP = D // 2

MM_BLK = 1000


def _mm_body(x_ref, w_ref, o_ref):
    o = jnp.dot(x_ref[...], w_ref[...], preferred_element_type=jnp.float32)
    o_ref[...] = o.astype(jnp.bfloat16)


def _matmul(x, W):
    return pl.pallas_call(
        _mm_body,
        grid=(N_NODES // MM_BLK,),
        in_specs=[
            pl.BlockSpec((MM_BLK, D), lambda i: (i, 0)),
            pl.BlockSpec((D, D), lambda i: (0, 0)),
        ],
        out_specs=pl.BlockSpec((MM_BLK, D), lambda i: (i, 0)),
        out_shape=jax.ShapeDtypeStruct((N_NODES, D), jnp.bfloat16),
    )(x, W)


_mesh_b = plsc.VectorSubcoreMesh(core_axis_name="c", subcore_axis_name="s",
                                 num_cores=1, num_subcores=NS)


@functools.partial(
    pl.kernel,
    out_type=jax.ShapeDtypeStruct((NS, BBLK, BE), jnp.float32),
    mesh=_mesh_b,
    scratch_types=[
        pltpu.VMEM((BBLK, BE), jnp.int32),
        pltpu.VMEM((BBLK, BE), jnp.int32),
        pltpu.VMEM((BBLK, BE), jnp.float32),
        pltpu.VMEM((DR, DC), jnp.float32),
        pltpu.VMEM((DRT, DC), jnp.float32),
        pltpu.VMEM((DRT, DC), jnp.float32),
        pltpu.VMEM_SHARED((NS, DR, DC), jnp.float32),
        pltpu.VMEM_SHARED((DR, DC), jnp.float32),
    ],
    compiler_params=pltpu.CompilerParams(needs_layout_passes=False),
)
def _sc_norm(src_hbm, dst_hbm, ew_hbm, nrm_hbm,
             src2, dst2, ew2, dis_v, dtmp, htmp, sh_hists, sh_dis):
    s = lax.axis_index("s")
    zeros = jnp.zeros((L,), jnp.float32)

    pltpu.sync_copy(src_hbm.at[s], src2)
    pltpu.sync_copy(dst_hbm.at[s], dst2)
    pltpu.sync_copy(ew_hbm.at[s], ew2)

    def _zhist(r, _):
        for j in range(DC // L):
            dis_v[r, pl.ds(j * L, L)] = zeros
        return 0
    lax.fori_loop(0, DR, _zhist, 0)

    def _deg(g, _):
        for j in range(BE // L):
            dv = dst2[g, pl.ds(j * L, L)]
            ev = ew2[g, pl.ds(j * L, L)]
            plsc.addupdate_scatter(dis_v, [dv >> 8, dv & 255], ev)
        return 0
    lax.fori_loop(0, BBLK, _deg, 0)
    pltpu.sync_copy(dis_v, sh_hists.at[s])
    plsc.subcore_barrier()

    pltpu.sync_copy(sh_hists.at[0, pl.ds(s * DRT, DRT)], dtmp)
    for p in range(1, NS):
        pltpu.sync_copy(sh_hists.at[p, pl.ds(s * DRT, DRT)], htmp)
        def _accum(r, _):
            for j in range(DC // L):
                dtmp[r, pl.ds(j * L, L)] = (dtmp[r, pl.ds(j * L, L)]
                                            + htmp[r, pl.ds(j * L, L)])
            return 0
        lax.fori_loop(0, DRT, _accum, 0)

    def _rsqrt(k, _):
        r = k // (DC // L)
        j16 = (k % (DC // L)) * L
        d = dtmp[r, pl.ds(j16, L)]
        dp = jnp.where(d > 0.0, d, 1.0)
        s0 = 0.5 * (1.0 + dp)
        def _nw(_i, s_c):
            return 0.5 * (s_c + dp / s_c)
        s0 = lax.fori_loop(0, 30, _nw, s0)
        dtmp[r, pl.ds(j16, L)] = jnp.where(d > 0.0, 1.0 / s0, 0.0)
        return 0
    lax.fori_loop(0, DRT * DC // L, _rsqrt, 0)
    pltpu.sync_copy(dtmp, sh_dis.at[pl.ds(s * DRT, DRT)])
    plsc.subcore_barrier()

    pltpu.sync_copy(sh_dis, dis_v)

    def _norm(g, _):
        for j in range(BE // L):
            sv = src2[g, pl.ds(j * L, L)]
            dv = dst2[g, pl.ds(j * L, L)]
            ev = ew2[g, pl.ds(j * L, L)]
            nm = plsc.load_gather(dis_v, [sv >> 8, sv & 255]) * ev \
                * plsc.load_gather(dis_v, [dv >> 8, dv & 255])
            ew2[g, pl.ds(j * L, L)] = nm
        return 0
    lax.fori_loop(0, BBLK, _norm, 0)
    pltpu.sync_copy(ew2, nrm_hbm.at[s])


_mesh_c = plsc.VectorSubcoreMesh(core_axis_name="c", subcore_axis_name="s",
                                 num_cores=NC, num_subcores=NS)


@functools.partial(
    pl.kernel,
    out_type=jax.ShapeDtypeStruct((N_NODES, D), jnp.float32),
    mesh=_mesh_c,
    scratch_types=[
        pltpu.VMEM((CH,), jnp.int32),
        pltpu.VMEM((CH,), jnp.int32),
        pltpu.VMEM((CH,), jnp.float32),
        pltpu.VMEM((CH + CE,), jnp.int32),
        pltpu.VMEM((CH + CE,), jnp.int32),
        pltpu.VMEM((CH + CE,), jnp.float32),
        pltpu.VMEM((1, CE), jnp.int32),
        pltpu.VMEM((CE, DP), jnp.int32),
        pltpu.VMEM((ROWS_PER_W, D), jnp.float32),
        pltpu.SemaphoreType.DMA,
    ],
    compiler_params=pltpu.CompilerParams(needs_layout_passes=False),
)
def _sc_scatter(src_hbm, dst_hbm, nrm_hbm, h_hbm, out_hbm,
                dstc, srcc, nmc, csrc, cdl, cnm, bidx2, rows0, acc, sem0):
    c = lax.axis_index("c")
    s = lax.axis_index("s")
    w = c * NS + s
    wlo = w * ROWS_PER_W
    zeros = jnp.zeros((L,), jnp.float32)
    zeros_i = jnp.zeros((L,), jnp.int32)
    iota = lax.iota(jnp.int32, L)

    def _zacc(r, _):
        for j in range(D // L):
            acc[r, pl.ds(j * L, L)] = zeros
        return 0
    lax.fori_loop(0, ROWS_PER_W, _zacc, 0)

    def _stage_issue(blk, par, rows, sem):
        for t in range(CE // L):
            bidx2[par, pl.ds(t * L, L)] = csrc[pl.ds(blk * CE + t * L, L)]
        pltpu.async_copy(h_hbm.at[bidx2.at[par]], rows, sem)

    def _process(blk, rows):
        def _qgrp(q, _4):
            dlv = cdl[pl.ds(blk * CE + q * L, L)]
            nmv = cnm[pl.ds(blk * CE + q * L, L)]
            rq = iota + (q * L)
            def _cols(cc, _3):
                bc0 = lax.broadcast(cc * L, (L,))
                for r in range(L):
                    colv = bc0 + ((iota + r) & (L - 1))
                    word = plsc.load_gather(rows, [rq, colv])
                    va = plsc.bitcast(word << 16, jnp.float32)
                    vb = plsc.bitcast(word & jnp.int32(-65536), jnp.float32)
                    oc = colv * 2
                    plsc.addupdate_scatter(acc, [dlv, oc], va * nmv)
                    plsc.addupdate_scatter(acc, [dlv, oc + 1], vb * nmv)
                return 0
            lax.fori_loop(0, DP // L, _cols, 0)
            return 0
        lax.fori_loop(0, CE // L, _qgrp, 0)

    def _chunk(k, cnt0):
        def _load_scan(c0):
            base = k * CH
            d1 = pltpu.async_copy(dst_hbm.at[pl.ds(base, CH)], dstc, sem0)
            d2 = pltpu.async_copy(src_hbm.at[pl.ds(base, CH)], srcc, sem0)
            d3 = pltpu.async_copy(nrm_hbm.at[pl.ds(base, CH)], nmc, sem0)
            d1.wait()
            d2.wait()
            d3.wait()

            def _scan(g, cnt):
                dv = dstc[pl.ds(g * L, L)]
                own = ((dv * 6554) >> 21) == w
                plsc.store_compressed(csrc.at[pl.ds(cnt, L)],
                                      srcc[pl.ds(g * L, L)], mask=own)
                plsc.store_compressed(cdl.at[pl.ds(cnt, L)], dv - wlo,
                                      mask=own)
                plsc.store_compressed(cnm.at[pl.ds(cnt, L)],
                                      nmc[pl.ds(g * L, L)], mask=own)
                pc = plsc.all_reduce_population_count(own)
                return cnt + pc[0]
            return lax.fori_loop(0, CH // L, _scan, c0)

        def _tail_pad(c0):
            for t in range(CE // L):
                csrc[pl.ds(c0 + t * L, L)] = zeros_i
                cdl[pl.ds(c0 + t * L, L)] = zeros_i
                cnm[pl.ds(c0 + t * L, L)] = zeros
            return jnp.where(c0 > 0, jnp.int32(CE), jnp.int32(0))

        cnt = lax.cond(k < NCH, _load_scan, _tail_pad, cnt0)
        nbf = cnt // CE

        def _blk(b, _2):
            _stage_issue(b, 0, rows0, sem0)
            pltpu.make_async_copy(h_hbm.at[bidx2.at[0]], rows0, sem0).wait()
            _process(b, rows0)
            return 0
        lax.fori_loop(0, nbf, _blk, 0)

        lo = cnt - nbf * CE
        @pl.when(nbf > 0)
        def _():
            for t in range(CE // L):
                @pl.when(t * L < lo)
                def _():
                    off = nbf * CE + t * L
                    csrc[pl.ds(t * L, L)] = csrc[pl.ds(off, L)]
                    cdl[pl.ds(t * L, L)] = cdl[pl.ds(off, L)]
                    cnm[pl.ds(t * L, L)] = cnm[pl.ds(off, L)]
        return lo
    lax.fori_loop(0, NCH + 1, _chunk, jnp.int32(0))

    def _relu(r, _):
        for j in range(D // L):
            v = acc[r, pl.ds(j * L, L)]
            acc[r, pl.ds(j * L, L)] = jnp.maximum(v, 0.0) + v
        return 0
    lax.fori_loop(0, ROWS_PER_W, _relu, 0)

    @pl.when(w < NW - 1)
    def _():
        pltpu.sync_copy(acc, out_hbm.at[pl.ds(wlo, ROWS_PER_W)])

    @pl.when(w == NW - 1)
    def _():
        last = N_NODES - (NW - 1) * ROWS_PER_W
        pltpu.sync_copy(acc.at[pl.ds(0, last)],
                        out_hbm.at[pl.ds(wlo, last)])


def kernel(x, edge_index, edge_weights, W):
    src = edge_index[0].astype(jnp.int32)
    dst = edge_index[1].astype(jnp.int32)
    ew = edge_weights.astype(jnp.float32)

    h = _matmul(x, W)
    hp = jax.lax.bitcast_convert_type(h.reshape(N_NODES, D // 2, 2),
                                      jnp.int32)
    nrm = _sc_norm(src.reshape(NS, BBLK, BE), dst.reshape(NS, BBLK, BE),
                   ew.reshape(NS, BBLK, BE))
    return _sc_scatter(src, dst, nrm.reshape(-1), hp)
